# baseline (speedup 1.0000x reference)
import functools

import jax
import jax.numpy as jnp
from jax import lax
from jax.experimental import pallas as pl
from jax.experimental.pallas import tpu as pltpu
from jax.experimental.pallas import tpu_sc as plsc

NC = 2
NS = 16
CHUNK = 128
NBUF = 7
SDEP = 4


def _cdiv(a, b):
    return -(-a // b)


def _make_deg_kernel(n_nodes, nd, n_edges):
    epw = n_edges // (NC * NS)
    ch = _cdiv(epw, CHUNK)
    rows_pt = nd // NS
    mesh = plsc.VectorSubcoreMesh(core_axis_name="c", subcore_axis_name="s")

    @functools.partial(
        pl.kernel,
        out_type=jax.ShapeDtypeStruct((2, NC, nd), jnp.float32),
        mesh=mesh,
        compiler_params=pltpu.CompilerParams(use_tc_tiling_on_sc=False),
        scratch_types=[
            pltpu.VMEM((ch, CHUNK), jnp.int32),
            pltpu.VMEM((ch, CHUNK), jnp.int32),
            pltpu.VMEM((CHUNK,), jnp.float32),
            pltpu.VMEM((rows_pt,), jnp.float32),
            pltpu.VMEM_SHARED((nd,), jnp.float32),
            pltpu.VMEM_SHARED((nd,), jnp.float32),
        ],
    )
    def deg_k(src_hbm, dst_hbm, zeros_hbm, ones_hbm, out_hbm,
              sidx, didx, obuf, zbuf, acc_o, acc_i):
        c = lax.axis_index("c")
        s = lax.axis_index("s")
        wid = c * NS + s
        pltpu.sync_copy(zeros_hbm, zbuf)
        pltpu.sync_copy(zbuf, acc_o.at[pl.ds(s * rows_pt, rows_pt)])
        pltpu.sync_copy(zbuf, acc_i.at[pl.ds(s * rows_pt, rows_pt)])
        pltpu.sync_copy(ones_hbm, obuf)
        pltpu.sync_copy(src_hbm.at[wid], sidx)
        pltpu.sync_copy(dst_hbm.at[wid], didx)
        plsc.subcore_barrier()

        def chunk(j, carry):
            pltpu.sync_copy(obuf, acc_o.at[sidx.at[j]], add=True)
            pltpu.sync_copy(obuf, acc_i.at[didx.at[j]], add=True)
            return carry

        lax.fori_loop(0, ch, chunk, 0)
        plsc.subcore_barrier()
        pltpu.sync_copy(acc_o.at[pl.ds(s * rows_pt, rows_pt)], zbuf)
        pltpu.sync_copy(zbuf, out_hbm.at[0].at[c].at[pl.ds(s * rows_pt, rows_pt)])
        pltpu.sync_copy(acc_i.at[pl.ds(s * rows_pt, rows_pt)], zbuf)
        pltpu.sync_copy(zbuf, out_hbm.at[1].at[c].at[pl.ds(s * rows_pt, rows_pt)])

    return deg_k


def _pipelined_pass(sidx, didx, tbl, rows, acc, gsem, ssem, ch):
    for b in range(NBUF):
        pltpu.async_copy(tbl.at[sidx.at[b]], rows.at[b], gsem)

    def chunk(j, carry):
        b = lax.rem(j, NBUF)
        pltpu.make_async_copy(tbl.at[sidx.at[j]], rows.at[b], gsem).wait()
        pltpu.async_copy(rows.at[b], acc.at[didx.at[j]], ssem, add=True)

        @pl.when(j >= SDEP)
        def _():
            jn = j - SDEP + NBUF
            bn = lax.rem(jn, NBUF)
            pltpu.make_async_copy(rows.at[bn], acc.at[didx.at[j]], ssem).wait()

            @pl.when(jn < ch)
            def _():
                pltpu.async_copy(tbl.at[sidx.at[jn]], rows.at[bn], gsem)

        return carry

    lax.fori_loop(0, ch, chunk, 0)
    for _ in range(SDEP):
        pltpu.make_async_copy(rows.at[0], acc.at[didx.at[0]], ssem).wait()


def _make_stage1_kernel(nd, n_edges):
    ch = _cdiv(n_edges // NS, CHUNK)
    rows_pt = nd // NS
    mesh = plsc.VectorSubcoreMesh(core_axis_name="c", subcore_axis_name="s")

    @functools.partial(
        pl.kernel,
        out_type=[jax.ShapeDtypeStruct((NC, nd, 24), jnp.float32),
                  jax.ShapeDtypeStruct((NC, nd, 16), jnp.float32)],
        mesh=mesh,
        compiler_params=pltpu.CompilerParams(use_tc_tiling_on_sc=False),
        scratch_types=[
            pltpu.VMEM((ch, CHUNK), jnp.int32),
            pltpu.VMEM((ch, CHUNK), jnp.int32),
            pltpu.VMEM((NBUF, CHUNK, 24), jnp.float32),
            pltpu.VMEM((rows_pt, 24), jnp.float32),
            pltpu.VMEM((rows_pt, 16), jnp.float32),
            pltpu.VMEM((rows_pt, 24), jnp.float32),
            pltpu.VMEM((rows_pt, 16), jnp.float32),
            pltpu.VMEM_SHARED((nd, 24), jnp.float32),
            pltpu.SemaphoreType.DMA,
            pltpu.SemaphoreType.DMA,
        ],
    )
    def k(z0_hbm, wvx_hbm, src_hbm, dst_hbm, z24_hbm, s1_hbm, a1_hbm,
          sidx, didx, rows, sbuf, wvbuf, ybuf, abuf, acc, gsem, ssem):
        c = lax.axis_index("c")
        s = lax.axis_index("s")
        neg = c == 0
        rs = pl.ds(s * rows_pt, rows_pt)
        pltpu.sync_copy(src_hbm.at[s], sidx)
        pltpu.sync_copy(dst_hbm.at[s], didx)
        pltpu.sync_copy(wvx_hbm.at[rs], wvbuf)
        pltpu.sync_copy(z24_hbm, ybuf)
        pltpu.sync_copy(ybuf, acc.at[rs])
        pltpu.sync_copy(z0_hbm.at[c].at[rs], sbuf)
        plsc.subcore_barrier()
        _pipelined_pass(sidx, didx, z0_hbm.at[c], rows, acc, gsem, ssem, ch)
        plsc.subcore_barrier()
        pltpu.sync_copy(acc.at[rs], ybuf)

        def u1(r, carry):
            wv = wvbuf[r, pl.ds(0, 16)]
            va = sbuf[r, pl.ds(0, 16)]
            vb = sbuf[r, pl.ds(8, 16)]
            ya = ybuf[r, pl.ds(0, 16)]
            yb = ybuf[r, pl.ds(8, 16)]
            na = jnp.where(neg, va - wv * ya, wv * ya)
            nb = jnp.where(neg, vb - wv * yb, wv * yb)
            sbuf[r, pl.ds(0, 16)] = na
            sbuf[r, pl.ds(8, 16)] = nb
            abuf[r, pl.ds(0, 16)] = nb
            return carry

        lax.fori_loop(0, rows_pt, u1, 0)
        pltpu.sync_copy(sbuf, s1_hbm.at[c].at[rs])
        pltpu.sync_copy(abuf, a1_hbm.at[c].at[rs])

    return k


def _make_stage23_kernel(nd, n_edges):
    ch = _cdiv(n_edges // NS, CHUNK)
    rows_pt = nd // NS
    mesh = plsc.VectorSubcoreMesh(core_axis_name="c", subcore_axis_name="s")

    @functools.partial(
        pl.kernel,
        out_type=jax.ShapeDtypeStruct((NC, nd, 24), jnp.float32),
        mesh=mesh,
        compiler_params=pltpu.CompilerParams(use_tc_tiling_on_sc=False),
        scratch_types=[
            pltpu.VMEM((ch, CHUNK), jnp.int32),
            pltpu.VMEM((ch, CHUNK), jnp.int32),
            pltpu.VMEM((NBUF, CHUNK, 16), jnp.float32),
            pltpu.VMEM((rows_pt, 24), jnp.float32),
            pltpu.VMEM((rows_pt, 16), jnp.float32),
            pltpu.VMEM((rows_pt, 16), jnp.float32),
            pltpu.VMEM((rows_pt, 16), jnp.float32),
            pltpu.VMEM_SHARED((nd, 16), jnp.float32),
            pltpu.VMEM_SHARED((nd, 16), jnp.float32),
            pltpu.SemaphoreType.DMA,
            pltpu.SemaphoreType.DMA,
        ],
    )
    def k(s1_hbm, a1_hbm, wvx_hbm, src_hbm, dst_hbm, z16_hbm, f_hbm,
          sidx, didx, rows, sbuf, wvbuf, ybuf, abuf, t16, acc, gsem, ssem):
        c = lax.axis_index("c")
        s = lax.axis_index("s")
        neg = c == 0
        rs = pl.ds(s * rows_pt, rows_pt)
        lanes = lax.iota(jnp.int32, 16)
        pltpu.sync_copy(src_hbm.at[s], sidx)
        pltpu.sync_copy(dst_hbm.at[s], didx)
        pltpu.sync_copy(wvx_hbm.at[rs], wvbuf)
        pltpu.sync_copy(z16_hbm, ybuf)
        pltpu.sync_copy(ybuf, acc.at[rs])
        pltpu.sync_copy(s1_hbm.at[c].at[rs], sbuf)
        plsc.subcore_barrier()
        _pipelined_pass(sidx, didx, a1_hbm.at[c], rows, acc, gsem, ssem, ch)
        plsc.subcore_barrier()
        pltpu.sync_copy(acc.at[rs], ybuf)

        def u2(r, carry):
            wv = wvbuf[r, pl.ds(0, 16)]
            vb = sbuf[r, pl.ds(8, 16)]
            yv = ybuf[r, pl.ds(0, 16)]
            nb = jnp.where(neg, vb - wv * yv, wv * yv)
            sbuf[r, pl.ds(8, 16)] = nb
            abuf[r, pl.ds(0, 16)] = jnp.where(lanes >= 8, nb, 0.0)
            return carry

        lax.fori_loop(0, rows_pt, u2, 0)
        pltpu.sync_copy(abuf, t16.at[rs])
        pltpu.sync_copy(z16_hbm, ybuf)
        pltpu.sync_copy(ybuf, acc.at[rs])
        plsc.subcore_barrier()
        _pipelined_pass(sidx, didx, t16, rows, acc, gsem, ssem, ch)
        plsc.subcore_barrier()
        pltpu.sync_copy(acc.at[rs], ybuf)

        def u3(r, carry):
            wv = wvbuf[r, pl.ds(0, 16)]
            va = sbuf[r, pl.ds(0, 16)]
            vb = sbuf[r, pl.ds(8, 16)]
            yv = ybuf[r, pl.ds(0, 16)]
            nb = jnp.where(neg, vb - wv * yv,
                           jnp.where(lanes >= 8, wv * yv, vb))
            sbuf[r, pl.ds(0, 16)] = jnp.maximum(va, 0.0)
            sbuf[r, pl.ds(8, 16)] = jnp.maximum(nb, 0.0)
            return carry

        lax.fori_loop(0, rows_pt, u3, 0)
        pltpu.sync_copy(sbuf, f_hbm.at[c].at[rs])

    return k


def _make_stage4_kernel(nd, n_edges):
    ch = _cdiv(n_edges // NS, CHUNK)
    rows_pt = nd // NS
    mesh = plsc.VectorSubcoreMesh(core_axis_name="c", subcore_axis_name="s")

    @functools.partial(
        pl.kernel,
        out_type=jax.ShapeDtypeStruct((NC, nd, 24), jnp.float32),
        mesh=mesh,
        compiler_params=pltpu.CompilerParams(use_tc_tiling_on_sc=False),
        scratch_types=[
            pltpu.VMEM((ch, CHUNK), jnp.int32),
            pltpu.VMEM((ch, CHUNK), jnp.int32),
            pltpu.VMEM((NBUF, CHUNK, 24), jnp.float32),
            pltpu.VMEM((rows_pt, 24), jnp.float32),
            pltpu.VMEM_SHARED((nd, 24), jnp.float32),
            pltpu.SemaphoreType.DMA,
            pltpu.SemaphoreType.DMA,
        ],
    )
    def k(f_hbm, src_hbm, dst_hbm, z24_hbm, out_hbm,
          sidx, didx, rows, ybuf, acc, gsem, ssem):
        c = lax.axis_index("c")
        s = lax.axis_index("s")
        rs = pl.ds(s * rows_pt, rows_pt)
        pltpu.sync_copy(src_hbm.at[s], sidx)
        pltpu.sync_copy(dst_hbm.at[s], didx)
        pltpu.sync_copy(z24_hbm, ybuf)
        pltpu.sync_copy(ybuf, acc.at[rs])
        plsc.subcore_barrier()
        _pipelined_pass(sidx, didx, f_hbm.at[c], rows, acc, gsem, ssem, ch)
        plsc.subcore_barrier()
        pltpu.sync_copy(acc.at[rs], ybuf)
        pltpu.sync_copy(ybuf, out_hbm.at[c].at[rs])

    return k


def kernel(x, edge_index, W_hyb, b_hyb, W_res, b_res):
    n = x.shape[0]
    e = edge_index.shape[1]
    d_in_dim = x.shape[1]
    hid = W_hyb.shape[2]
    nch = W_hyb.shape[0]
    nd = _cdiv(n + 1, NS * 16) * NS * 16
    rows_pt = nd // NS

    src = edge_index[0].astype(jnp.int32)
    dst = edge_index[1].astype(jnp.int32)

    epw = e // (NC * NS)
    ch_d = _cdiv(epw, CHUNK)
    pad_d = jnp.full((NC * NS, ch_d * CHUNK - epw), n, jnp.int32)
    src_d = jnp.concatenate([src.reshape(NC * NS, epw), pad_d], 1).reshape(
        NC * NS, ch_d, CHUNK)
    dst_d = jnp.concatenate([dst.reshape(NC * NS, epw), pad_d], 1).reshape(
        NC * NS, ch_d, CHUNK)
    epc = e // NS
    ch_p = _cdiv(epc, CHUNK)
    src_p = jnp.concatenate(
        [src.reshape(NS, epc),
         jnp.zeros((NS, ch_p * CHUNK - epc), jnp.int32)], 1).reshape(
        NS, ch_p, CHUNK)
    dst_p = jnp.concatenate(
        [dst.reshape(NS, epc),
         jnp.full((NS, ch_p * CHUNK - epc), n, jnp.int32)], 1).reshape(
        NS, ch_p, CHUNK)

    z640_1 = jnp.zeros((rows_pt,), jnp.float32)
    ones_c = jnp.ones((CHUNK,), jnp.float32)
    z24 = jnp.zeros((rows_pt, 24), jnp.float32)
    z16 = jnp.zeros((rows_pt, 16), jnp.float32)

    Wcat = jnp.transpose(W_hyb, (1, 0, 2)).reshape(d_in_dim, nch * hid)
    bcat = b_hyb.reshape(nch * hid)

    deg = _make_deg_kernel(n, nd, e)(src_d, dst_d, z640_1, ones_c)

    def prep_body(x_ref, wc_ref, bc_ref, deg_ref, z0_ref, wvx_ref, vv_ref):
        d_out = deg_ref[0, 0, :n] + deg_ref[0, 1, :n]
        d_inn = deg_ref[1, 0, :n] + deg_ref[1, 1, :n]
        u = lax.rsqrt(jnp.maximum(d_out, 1.0))
        v = lax.rsqrt(jnp.maximum(d_inn, 1.0))
        h0 = jnp.dot(x_ref[...], wc_ref[...],
                     preferred_element_type=jnp.float32) + bc_ref[...][None, :]
        z0 = h0 * u[:, None]
        zpad = jnp.zeros((nd - n, 24), jnp.float32)
        z0_ref[0] = jnp.concatenate([z0[:, :24], zpad], axis=0)
        z0_ref[1] = jnp.concatenate([z0[:, 24:], zpad], axis=0)
        wvx = jnp.broadcast_to((u * v)[:, None], (n, 16))
        wvx_ref[...] = jnp.concatenate(
            [wvx, jnp.zeros((nd - n, 16), jnp.float32)], axis=0)
        vv_ref[...] = v[:, None]

    z0, wvx, vv = pl.pallas_call(
        prep_body,
        out_shape=[jax.ShapeDtypeStruct((NC, nd, 24), jnp.float32),
                   jax.ShapeDtypeStruct((nd, 16), jnp.float32),
                   jax.ShapeDtypeStruct((n, 1), jnp.float32)],
    )(x, Wcat, bcat, deg)

    s1, a1 = _make_stage1_kernel(nd, e)(z0, wvx, src_p, dst_p, z24)
    f = _make_stage23_kernel(nd, e)(s1, a1, wvx, src_p, dst_p, z16)
    y4 = _make_stage4_kernel(nd, e)(f, src_p, dst_p, z24)

    def fin_body(y_ref, v_ref, wr_ref, br_ref, o_ref):
        h = jnp.concatenate([y_ref[0, :n, :], y_ref[1, :n, :]], axis=1)
        h = h * v_ref[...]
        o_ref[...] = jnp.dot(h, wr_ref[...],
                             preferred_element_type=jnp.float32) + br_ref[...][None, :]

    out = pl.pallas_call(
        fin_body,
        out_shape=jax.ShapeDtypeStruct((n, W_res.shape[1]), jnp.float32),
    )(y4, vv, W_res, b_res)
    return out


# speedup vs baseline: 1.1623x; 1.1623x over previous
import functools

import jax
import jax.numpy as jnp
from jax import lax
from jax.experimental import pallas as pl
from jax.experimental.pallas import tpu as pltpu
from jax.experimental.pallas import tpu_sc as plsc

NC = 2
NS = 16
CHUNK = 128
NBUF = 7
SDEP = 2


def _cdiv(a, b):
    return -(-a // b)


def _make_deg_kernel(n_nodes, nd, n_edges):
    epw = n_edges // (NC * NS)
    ch = _cdiv(epw, CHUNK)
    rows_pt = nd // NS
    mesh = plsc.VectorSubcoreMesh(core_axis_name="c", subcore_axis_name="s")

    @functools.partial(
        pl.kernel,
        out_type=jax.ShapeDtypeStruct((2, NC, nd), jnp.float32),
        mesh=mesh,
        compiler_params=pltpu.CompilerParams(use_tc_tiling_on_sc=False),
        scratch_types=[
            pltpu.VMEM((ch, CHUNK), jnp.int32),
            pltpu.VMEM((ch, CHUNK), jnp.int32),
            pltpu.VMEM((CHUNK,), jnp.float32),
            pltpu.VMEM((rows_pt,), jnp.float32),
            pltpu.VMEM_SHARED((nd,), jnp.float32),
            pltpu.VMEM_SHARED((nd,), jnp.float32),
        ],
    )
    def deg_k(src_hbm, dst_hbm, zeros_hbm, ones_hbm, out_hbm,
              sidx, didx, obuf, zbuf, acc_o, acc_i):
        c = lax.axis_index("c")
        s = lax.axis_index("s")
        wid = c * NS + s
        pltpu.sync_copy(zeros_hbm, zbuf)
        pltpu.sync_copy(zbuf, acc_o.at[pl.ds(s * rows_pt, rows_pt)])
        pltpu.sync_copy(zbuf, acc_i.at[pl.ds(s * rows_pt, rows_pt)])
        pltpu.sync_copy(ones_hbm, obuf)
        pltpu.sync_copy(src_hbm.at[wid], sidx)
        pltpu.sync_copy(dst_hbm.at[wid], didx)
        plsc.subcore_barrier()

        def chunk(j, carry):
            pltpu.sync_copy(obuf, acc_o.at[sidx.at[j]], add=True)
            pltpu.sync_copy(obuf, acc_i.at[didx.at[j]], add=True)
            return carry

        lax.fori_loop(0, ch, chunk, 0)
        plsc.subcore_barrier()
        pltpu.sync_copy(acc_o.at[pl.ds(s * rows_pt, rows_pt)], zbuf)
        pltpu.sync_copy(zbuf, out_hbm.at[0].at[c].at[pl.ds(s * rows_pt, rows_pt)])
        pltpu.sync_copy(acc_i.at[pl.ds(s * rows_pt, rows_pt)], zbuf)
        pltpu.sync_copy(zbuf, out_hbm.at[1].at[c].at[pl.ds(s * rows_pt, rows_pt)])

    return deg_k


def _pipelined_pass(sidx, didx, tbl, rows, acc, gsem, ssem, ch):
    for b in range(NBUF):
        pltpu.async_copy(tbl.at[sidx.at[b]], rows.at[b], gsem)

    def chunk(j, carry):
        b = lax.rem(j, NBUF)
        pltpu.make_async_copy(tbl.at[sidx.at[j]], rows.at[b], gsem).wait()
        pltpu.async_copy(rows.at[b], acc.at[didx.at[j]], ssem, add=True)

        @pl.when(j >= SDEP)
        def _():
            jn = j - SDEP + NBUF
            bn = lax.rem(jn, NBUF)
            pltpu.make_async_copy(rows.at[bn], acc.at[didx.at[j]], ssem).wait()

            @pl.when(jn < ch)
            def _():
                pltpu.async_copy(tbl.at[sidx.at[jn]], rows.at[bn], gsem)

        return carry

    lax.fori_loop(0, ch, chunk, 0)
    for _ in range(SDEP):
        pltpu.make_async_copy(rows.at[0], acc.at[didx.at[0]], ssem).wait()


def _make_stage1_kernel(nd, n_edges):
    ch = _cdiv(n_edges // NS, CHUNK)
    rows_pt = nd // NS
    mesh = plsc.VectorSubcoreMesh(core_axis_name="c", subcore_axis_name="s")

    @functools.partial(
        pl.kernel,
        out_type=[jax.ShapeDtypeStruct((NC, nd, 24), jnp.float32),
                  jax.ShapeDtypeStruct((NC, nd, 16), jnp.float32)],
        mesh=mesh,
        compiler_params=pltpu.CompilerParams(use_tc_tiling_on_sc=False),
        scratch_types=[
            pltpu.VMEM((ch, CHUNK), jnp.int32),
            pltpu.VMEM((ch, CHUNK), jnp.int32),
            pltpu.VMEM((NBUF, CHUNK, 24), jnp.float32),
            pltpu.VMEM((rows_pt, 24), jnp.float32),
            pltpu.VMEM((rows_pt, 16), jnp.float32),
            pltpu.VMEM((rows_pt, 24), jnp.float32),
            pltpu.VMEM((rows_pt, 16), jnp.float32),
            pltpu.VMEM_SHARED((nd, 24), jnp.float32),
            pltpu.SemaphoreType.DMA,
            pltpu.SemaphoreType.DMA,
        ],
    )
    def k(z0_hbm, wvx_hbm, src_hbm, dst_hbm, z24_hbm, s1_hbm, a1_hbm,
          sidx, didx, rows, sbuf, wvbuf, ybuf, abuf, acc, gsem, ssem):
        c = lax.axis_index("c")
        s = lax.axis_index("s")
        neg = c == 0
        rs = pl.ds(s * rows_pt, rows_pt)
        pltpu.sync_copy(src_hbm.at[s], sidx)
        pltpu.sync_copy(dst_hbm.at[s], didx)
        pltpu.sync_copy(wvx_hbm.at[rs], wvbuf)
        pltpu.sync_copy(z24_hbm, ybuf)
        pltpu.sync_copy(ybuf, acc.at[rs])
        pltpu.sync_copy(z0_hbm.at[c].at[rs], sbuf)
        plsc.subcore_barrier()
        _pipelined_pass(sidx, didx, z0_hbm.at[c], rows, acc, gsem, ssem, ch)
        plsc.subcore_barrier()
        pltpu.sync_copy(acc.at[rs], ybuf)

        def u1(r, carry):
            wv = wvbuf[r, pl.ds(0, 16)]
            va = sbuf[r, pl.ds(0, 16)]
            vb = sbuf[r, pl.ds(8, 16)]
            ya = ybuf[r, pl.ds(0, 16)]
            yb = ybuf[r, pl.ds(8, 16)]
            na = jnp.where(neg, va - wv * ya, wv * ya)
            nb = jnp.where(neg, vb - wv * yb, wv * yb)
            sbuf[r, pl.ds(0, 16)] = na
            sbuf[r, pl.ds(8, 16)] = nb
            abuf[r, pl.ds(0, 16)] = nb
            return carry

        lax.fori_loop(0, rows_pt, u1, 0)
        pltpu.sync_copy(sbuf, s1_hbm.at[c].at[rs])
        pltpu.sync_copy(abuf, a1_hbm.at[c].at[rs])

    return k


def _make_stage23_kernel(nd, n_edges):
    ch = _cdiv(n_edges // NS, CHUNK)
    rows_pt = nd // NS
    mesh = plsc.VectorSubcoreMesh(core_axis_name="c", subcore_axis_name="s")

    @functools.partial(
        pl.kernel,
        out_type=jax.ShapeDtypeStruct((NC, nd, 24), jnp.float32),
        mesh=mesh,
        compiler_params=pltpu.CompilerParams(use_tc_tiling_on_sc=False),
        scratch_types=[
            pltpu.VMEM((ch, CHUNK), jnp.int32),
            pltpu.VMEM((ch, CHUNK), jnp.int32),
            pltpu.VMEM((NBUF, CHUNK, 16), jnp.float32),
            pltpu.VMEM((rows_pt, 24), jnp.float32),
            pltpu.VMEM((rows_pt, 16), jnp.float32),
            pltpu.VMEM((rows_pt, 16), jnp.float32),
            pltpu.VMEM((rows_pt, 16), jnp.float32),
            pltpu.VMEM_SHARED((nd, 16), jnp.float32),
            pltpu.VMEM_SHARED((nd, 16), jnp.float32),
            pltpu.SemaphoreType.DMA,
            pltpu.SemaphoreType.DMA,
        ],
    )
    def k(s1_hbm, a1_hbm, wvx_hbm, src_hbm, dst_hbm, z16_hbm, f_hbm,
          sidx, didx, rows, sbuf, wvbuf, ybuf, abuf, t16, acc, gsem, ssem):
        c = lax.axis_index("c")
        s = lax.axis_index("s")
        neg = c == 0
        rs = pl.ds(s * rows_pt, rows_pt)
        lanes = lax.iota(jnp.int32, 16)
        pltpu.sync_copy(src_hbm.at[s], sidx)
        pltpu.sync_copy(dst_hbm.at[s], didx)
        pltpu.sync_copy(wvx_hbm.at[rs], wvbuf)
        pltpu.sync_copy(z16_hbm, ybuf)
        pltpu.sync_copy(ybuf, acc.at[rs])
        pltpu.sync_copy(s1_hbm.at[c].at[rs], sbuf)
        plsc.subcore_barrier()
        _pipelined_pass(sidx, didx, a1_hbm.at[c], rows, acc, gsem, ssem, ch)
        plsc.subcore_barrier()
        pltpu.sync_copy(acc.at[rs], ybuf)

        def u2(r, carry):
            wv = wvbuf[r, pl.ds(0, 16)]
            vb = sbuf[r, pl.ds(8, 16)]
            yv = ybuf[r, pl.ds(0, 16)]
            nb = jnp.where(neg, vb - wv * yv, wv * yv)
            sbuf[r, pl.ds(8, 16)] = nb
            abuf[r, pl.ds(0, 16)] = jnp.where(lanes >= 8, nb, 0.0)
            return carry

        lax.fori_loop(0, rows_pt, u2, 0)
        pltpu.sync_copy(abuf, t16.at[rs])
        pltpu.sync_copy(z16_hbm, ybuf)
        pltpu.sync_copy(ybuf, acc.at[rs])
        plsc.subcore_barrier()
        _pipelined_pass(sidx, didx, t16, rows, acc, gsem, ssem, ch)
        plsc.subcore_barrier()
        pltpu.sync_copy(acc.at[rs], ybuf)

        def u3(r, carry):
            wv = wvbuf[r, pl.ds(0, 16)]
            va = sbuf[r, pl.ds(0, 16)]
            vb = sbuf[r, pl.ds(8, 16)]
            yv = ybuf[r, pl.ds(0, 16)]
            nb = jnp.where(neg, vb - wv * yv,
                           jnp.where(lanes >= 8, wv * yv, vb))
            sbuf[r, pl.ds(0, 16)] = jnp.maximum(va, 0.0)
            sbuf[r, pl.ds(8, 16)] = jnp.maximum(nb, 0.0)
            return carry

        lax.fori_loop(0, rows_pt, u3, 0)
        pltpu.sync_copy(sbuf, f_hbm.at[c].at[rs])

    return k


def _make_stage4_kernel(nd, n_edges):
    ch = _cdiv(n_edges // NS, CHUNK)
    rows_pt = nd // NS
    mesh = plsc.VectorSubcoreMesh(core_axis_name="c", subcore_axis_name="s")

    @functools.partial(
        pl.kernel,
        out_type=jax.ShapeDtypeStruct((NC, nd, 24), jnp.float32),
        mesh=mesh,
        compiler_params=pltpu.CompilerParams(use_tc_tiling_on_sc=False),
        scratch_types=[
            pltpu.VMEM((ch, CHUNK), jnp.int32),
            pltpu.VMEM((ch, CHUNK), jnp.int32),
            pltpu.VMEM((NBUF, CHUNK, 24), jnp.float32),
            pltpu.VMEM((rows_pt, 24), jnp.float32),
            pltpu.VMEM_SHARED((nd, 24), jnp.float32),
            pltpu.SemaphoreType.DMA,
            pltpu.SemaphoreType.DMA,
        ],
    )
    def k(f_hbm, src_hbm, dst_hbm, z24_hbm, out_hbm,
          sidx, didx, rows, ybuf, acc, gsem, ssem):
        c = lax.axis_index("c")
        s = lax.axis_index("s")
        rs = pl.ds(s * rows_pt, rows_pt)
        pltpu.sync_copy(src_hbm.at[s], sidx)
        pltpu.sync_copy(dst_hbm.at[s], didx)
        pltpu.sync_copy(z24_hbm, ybuf)
        pltpu.sync_copy(ybuf, acc.at[rs])
        plsc.subcore_barrier()
        _pipelined_pass(sidx, didx, f_hbm.at[c], rows, acc, gsem, ssem, ch)
        plsc.subcore_barrier()
        pltpu.sync_copy(acc.at[rs], ybuf)
        pltpu.sync_copy(ybuf, out_hbm.at[c].at[rs])

    return k


def kernel(x, edge_index, W_hyb, b_hyb, W_res, b_res):
    n = x.shape[0]
    e = edge_index.shape[1]
    d_in_dim = x.shape[1]
    hid = W_hyb.shape[2]
    nch = W_hyb.shape[0]
    nd = _cdiv(n + 1, NS * 16) * NS * 16
    rows_pt = nd // NS

    src = edge_index[0].astype(jnp.int32)
    dst = edge_index[1].astype(jnp.int32)

    epw = e // (NC * NS)
    ch_d = _cdiv(epw, CHUNK)
    pad_d = jnp.full((NC * NS, ch_d * CHUNK - epw), n, jnp.int32)
    src_d = jnp.concatenate([src.reshape(NC * NS, epw), pad_d], 1).reshape(
        NC * NS, ch_d, CHUNK)
    dst_d = jnp.concatenate([dst.reshape(NC * NS, epw), pad_d], 1).reshape(
        NC * NS, ch_d, CHUNK)
    epc = e // NS
    ch_p = _cdiv(epc, CHUNK)
    src_p = jnp.concatenate(
        [src.reshape(NS, epc),
         jnp.zeros((NS, ch_p * CHUNK - epc), jnp.int32)], 1).reshape(
        NS, ch_p, CHUNK)
    dst_p = jnp.concatenate(
        [dst.reshape(NS, epc),
         jnp.full((NS, ch_p * CHUNK - epc), n, jnp.int32)], 1).reshape(
        NS, ch_p, CHUNK)

    z640_1 = jnp.zeros((rows_pt,), jnp.float32)
    ones_c = jnp.ones((CHUNK,), jnp.float32)
    z24 = jnp.zeros((rows_pt, 24), jnp.float32)
    z16 = jnp.zeros((rows_pt, 16), jnp.float32)

    Wcat = jnp.transpose(W_hyb, (1, 0, 2)).reshape(d_in_dim, nch * hid)
    bcat = b_hyb.reshape(nch * hid)

    deg = _make_deg_kernel(n, nd, e)(src_d, dst_d, z640_1, ones_c)

    def prep_body(x_ref, wc_ref, bc_ref, deg_ref, z0_ref, wvx_ref, vv_ref):
        d_out = deg_ref[0, 0, :n] + deg_ref[0, 1, :n]
        d_inn = deg_ref[1, 0, :n] + deg_ref[1, 1, :n]
        u = lax.rsqrt(jnp.maximum(d_out, 1.0))
        v = lax.rsqrt(jnp.maximum(d_inn, 1.0))
        h0 = jnp.dot(x_ref[...], wc_ref[...],
                     preferred_element_type=jnp.float32) + bc_ref[...][None, :]
        z0 = h0 * u[:, None]
        zpad = jnp.zeros((nd - n, 24), jnp.float32)
        z0_ref[0] = jnp.concatenate([z0[:, :24], zpad], axis=0)
        z0_ref[1] = jnp.concatenate([z0[:, 24:], zpad], axis=0)
        wvx = jnp.broadcast_to((u * v)[:, None], (n, 16))
        wvx_ref[...] = jnp.concatenate(
            [wvx, jnp.zeros((nd - n, 16), jnp.float32)], axis=0)
        vv_ref[...] = v[:, None]

    z0, wvx, vv = pl.pallas_call(
        prep_body,
        out_shape=[jax.ShapeDtypeStruct((NC, nd, 24), jnp.float32),
                   jax.ShapeDtypeStruct((nd, 16), jnp.float32),
                   jax.ShapeDtypeStruct((n, 1), jnp.float32)],
    )(x, Wcat, bcat, deg)

    s1, a1 = _make_stage1_kernel(nd, e)(z0, wvx, src_p, dst_p, z24)
    f = _make_stage23_kernel(nd, e)(s1, a1, wvx, src_p, dst_p, z16)
    y4 = _make_stage4_kernel(nd, e)(f, src_p, dst_p, z24)

    def fin_body(y_ref, v_ref, wr_ref, br_ref, o_ref):
        h = jnp.concatenate([y_ref[0, :n, :], y_ref[1, :n, :]], axis=1)
        h = h * v_ref[...]
        o_ref[...] = jnp.dot(h, wr_ref[...],
                             preferred_element_type=jnp.float32) + br_ref[...][None, :]

    out = pl.pallas_call(
        fin_body,
        out_shape=jax.ShapeDtypeStruct((n, W_res.shape[1]), jnp.float32),
    )(y4, vv, W_res, b_res)
    return out
